# baseline (speedup 1.0000x reference)
import jax
import jax.numpy as jnp
from jax import lax
from jax.experimental import pallas as pl
from jax.experimental.pallas import tpu as pltpu
from jax.experimental.pallas import tpu_sc as plsc

NUM_IMG = 128
VOCAB = 100000
PER_IMG = 100
NC, NS, L = 2, 16, 16
NW = NC * NS
ROWS_PER_W = NUM_IMG // NW
NVEC = 7
TAIL = PER_IMG - (NVEC - 1) * L
NCHUNK = PER_IMG + 1


def _hsum(x, lane):
    for s in (1, 2, 4, 8):
        rot = (lane + s) & (L - 1)
        x = x + x.at[rot].get(mode="promise_in_bounds", unique_indices=True)
    return x


def _sc_body(inp_hbm, tgt_hbm, out_hbm, tgt_v, table_v, w_v, chunks_v, ovec_v,
             sem_a, sem_b):
    cid = lax.axis_index("c")
    sid = lax.axis_index("s")
    wid = sid * NC + cid
    nt = ROWS_PER_W * PER_IMG
    pltpu.sync_copy(tgt_hbm.at[pl.ds(wid * nt, nt)], tgt_v.at[pl.ds(0, nt)])

    lane = lax.iota(jnp.int32, L)
    valid_tail = lane < TAIL
    all_true = lane < L
    sems = (sem_a, sem_b)

    acc_a = jnp.zeros((L,), jnp.float32)
    acc_b = jnp.zeros((L,), jnp.float32)
    mmax = jnp.zeros((L,), jnp.float32)

    for r in range(ROWS_PER_W):
        g = wid * ROWS_PER_W + r
        base = r * PER_IMG
        v = []
        for k in range(NVEC):
            vk = tgt_v[pl.ds(base + L * k, L)]
            if k == NVEC - 1:
                vk = jnp.where(valid_tail, vk, 0)
            v.append(vk)

        for k in range(NVEC):
            mask = valid_tail if k == NVEC - 1 else all_true
            plsc.store_scatter(table_v, [v[k]], lane + L * k, mask=mask)
        for k in range(NVEC):
            rb = plsc.load_gather(table_v, [v[k]])
            first = rb == (lane + L * k)
            if k == NVEC - 1:
                first = jnp.logical_and(first, valid_tail)
            w_v[r, pl.ds(L * k, L)] = jnp.where(first, 1.0, 0.0).astype(jnp.float32)

        nissue = [0, 0]
        for k in range(NVEC):
            nlane = TAIL if k == NVEC - 1 else L
            cvec = v[k] >> 4
            for j in range(nlane):
                c_s = cvec[j]
                slot = L * k + j
                pltpu.async_copy(
                    inp_hbm.at[g, pl.ds(c_s * L, L)],
                    chunks_v.at[slot], sems[slot % 2])
                nissue[slot % 2] += 1
        pltpu.async_copy(inp_hbm.at[g, pl.ds(0, L)],
                         chunks_v.at[PER_IMG], sems[0])
        nissue[0] += 1
        for q in range(2):
            for _ in range(nissue[q]):
                pltpu.make_async_copy(inp_hbm.at[g, pl.ds(0, L)],
                                      chunks_v.at[PER_IMG], sems[q]).wait()

        ssum = jnp.zeros((L,), jnp.float32)
        lens = jnp.zeros((L,), jnp.float32)
        for k in range(NVEC):
            slot = lane + L * k
            slot = jnp.where(slot < PER_IMG, slot, PER_IMG)
            vals = plsc.load_gather(chunks_v, [slot, v[k] & (L - 1)])
            wv = w_v[r, pl.ds(L * k, L)]
            ssum = ssum + vals * wv
            lens = lens + wv
        z = plsc.load_gather(
            chunks_v,
            [jnp.full((L,), PER_IMG, jnp.int32), jnp.zeros((L,), jnp.int32)])
        acc_a = acc_a + ssum - lens * z
        acc_b = acc_b + z
        mmax = jnp.maximum(mmax, _hsum(lens, lane))

    a_s = _hsum(acc_a, lane)
    b_s = _hsum(acc_b, lane) / float(L)
    ov = jnp.where(lane == 0, a_s,
                   jnp.where(lane == 1, b_s,
                             jnp.where(lane == 2, mmax, 0.0))).astype(jnp.float32)
    ovec_v[...] = ov
    pltpu.sync_copy(ovec_v, out_hbm.at[wid])


def _sc_partials(inp2d, tgt_flat):
    mesh = plsc.VectorSubcoreMesh(core_axis_name="c", subcore_axis_name="s",
                                  num_cores=NC, num_subcores=NS)
    f = pl.kernel(
        _sc_body,
        out_type=jax.ShapeDtypeStruct((NW, L), jnp.float32),
        mesh=mesh,
        compiler_params=pltpu.CompilerParams(needs_layout_passes=False),
        scratch_types=[
            pltpu.VMEM((ROWS_PER_W * PER_IMG + L,), jnp.int32),
            pltpu.VMEM((VOCAB,), jnp.int32),
            pltpu.VMEM((ROWS_PER_W, 8 * L), jnp.float32),
            pltpu.VMEM((NCHUNK + 3, L), jnp.float32),
            pltpu.VMEM((L,), jnp.float32),
            pltpu.SemaphoreType.DMA,
            pltpu.SemaphoreType.DMA,
        ],
    )
    return f(inp2d, tgt_flat)


def kernel(input, target):
    tgt_flat = target.reshape(-1)
    p = _sc_partials(input, tgt_flat)
    a = jnp.sum(p[:, 0])
    b = jnp.sum(p[:, 1])
    m = jnp.max(p[:, 2])
    return -a / (NUM_IMG * m) - b / NUM_IMG


# speedup vs baseline: 1.0585x; 1.0585x over previous
import jax
import jax.numpy as jnp
from jax import lax
from jax.experimental import pallas as pl
from jax.experimental.pallas import tpu as pltpu
from jax.experimental.pallas import tpu_sc as plsc

NUM_IMG = 128
VOCAB = 100000
PER_IMG = 100
NC, NS, L = 2, 16, 16
NW = NC * NS
ROWS_PER_W = NUM_IMG // NW
NVEC = 7
TAIL = PER_IMG - (NVEC - 1) * L
NCHUNK = PER_IMG + 1


def _hsum(x, lane):
    for s in (1, 2, 4, 8):
        rot = (lane + s) & (L - 1)
        x = x + x.at[rot].get(mode="promise_in_bounds", unique_indices=True)
    return x


def _sc_body(inp_hbm, tgt_hbm, out_hbm, tgt_v, table_v, w_v, chunks_v, ovec_v,
             sem_a, sem_b):
    cid = lax.axis_index("c")
    sid = lax.axis_index("s")
    wid = sid * NC + cid
    nt = ROWS_PER_W * PER_IMG
    pltpu.sync_copy(tgt_hbm.at[pl.ds(wid * nt, nt)], tgt_v.at[pl.ds(0, nt)])

    lane = lax.iota(jnp.int32, L)
    valid_tail = lane < TAIL
    all_true = lane < L
    sems = (sem_a, sem_b)

    acc_a = jnp.zeros((L,), jnp.float32)
    acc_b = jnp.zeros((L,), jnp.float32)
    mmax = jnp.zeros((L,), jnp.float32)

    for r in range(ROWS_PER_W):
        g = wid * ROWS_PER_W + r
        base = r * PER_IMG
        v = []
        for k in range(NVEC):
            vk = tgt_v[pl.ds(base + L * k, L)]
            if k == NVEC - 1:
                vk = jnp.where(valid_tail, vk, 0)
            v.append(vk)

        for k in range(NVEC):
            mask = valid_tail if k == NVEC - 1 else all_true
            plsc.store_scatter(table_v, [v[k]], lane + L * k, mask=mask)
        for k in range(NVEC):
            rb = plsc.load_gather(table_v, [v[k]])
            first = rb == (lane + L * k)
            if k == NVEC - 1:
                first = jnp.logical_and(first, valid_tail)
            w_v[r, pl.ds(L * k, L)] = jnp.where(first, 1.0, 0.0).astype(jnp.float32)

        nissue = [0, 0]
        for k in range(NVEC):
            nlane = TAIL if k == NVEC - 1 else L
            cvec = v[k] >> 7
            for j in range(nlane):
                c_s = cvec[j]
                slot = L * k + j
                pltpu.async_copy(
                    inp_hbm.at[g, pl.ds(c_s * 128, 128)],
                    chunks_v.at[slot], sems[slot % 2])
                nissue[slot % 2] += 1
        pltpu.async_copy(inp_hbm.at[g, pl.ds(0, 128)],
                         chunks_v.at[PER_IMG], sems[0])
        nissue[0] += 1
        for q in range(2):
            for _ in range(nissue[q]):
                pltpu.make_async_copy(inp_hbm.at[g, pl.ds(0, 128)],
                                      chunks_v.at[PER_IMG], sems[q]).wait()

        ssum = jnp.zeros((L,), jnp.float32)
        lens = jnp.zeros((L,), jnp.float32)
        for k in range(NVEC):
            slot = lane + L * k
            slot = jnp.where(slot < PER_IMG, slot, PER_IMG)
            vals = plsc.load_gather(chunks_v, [slot, v[k] & 127])
            wv = w_v[r, pl.ds(L * k, L)]
            ssum = ssum + vals * wv
            lens = lens + wv
        z = plsc.load_gather(
            chunks_v,
            [jnp.full((L,), PER_IMG, jnp.int32), jnp.zeros((L,), jnp.int32)])
        acc_a = acc_a + ssum - lens * z
        acc_b = acc_b + z
        mmax = jnp.maximum(mmax, _hsum(lens, lane))

    a_s = _hsum(acc_a, lane)
    b_s = _hsum(acc_b, lane) / float(L)
    ov = jnp.where(lane == 0, a_s,
                   jnp.where(lane == 1, b_s,
                             jnp.where(lane == 2, mmax, 0.0))).astype(jnp.float32)
    ovec_v[...] = ov
    pltpu.sync_copy(ovec_v, out_hbm.at[wid])


def _sc_partials(inp2d, tgt_flat):
    mesh = plsc.VectorSubcoreMesh(core_axis_name="c", subcore_axis_name="s",
                                  num_cores=NC, num_subcores=NS)
    f = pl.kernel(
        _sc_body,
        out_type=jax.ShapeDtypeStruct((NW, L), jnp.float32),
        mesh=mesh,
        compiler_params=pltpu.CompilerParams(needs_layout_passes=False),
        scratch_types=[
            pltpu.VMEM((ROWS_PER_W * PER_IMG + L,), jnp.int32),
            pltpu.VMEM((VOCAB,), jnp.int32),
            pltpu.VMEM((ROWS_PER_W, 8 * L), jnp.float32),
            pltpu.VMEM((NCHUNK + 3, 128), jnp.float32),
            pltpu.VMEM((L,), jnp.float32),
            pltpu.SemaphoreType.DMA,
            pltpu.SemaphoreType.DMA,
        ],
    )
    return f(inp2d, tgt_flat)


def kernel(input, target):
    tgt_flat = target.reshape(-1)
    p = _sc_partials(input, tgt_flat)
    a = jnp.sum(p[:, 0])
    b = jnp.sum(p[:, 1])
    m = jnp.max(p[:, 2])
    return -a / (NUM_IMG * m) - b / NUM_IMG
